# Initial kernel scaffold; baseline (speedup 1.0000x reference)
import jax
import jax.numpy as jnp
from jax.experimental import pallas as pl


def kernel(lprobs, scores_prev):
    raise NotImplementedError("write your pallas kernel here")


# speedup vs baseline: 4.1955x; 4.1955x over previous
import functools

import numpy as _np

import jax
import jax.numpy as jnp
from jax import lax
from jax.experimental import pallas as pl
from jax.experimental.pallas import tpu as pltpu
from jax.experimental.pallas import tpu_sc as plsc

_BSZ = 64
_BEAM = 4
_VOCAB = 100000
_PAD = 1
_EOS = 2
_CAND = 8
_BLK = 800
_NBLK = _VOCAB // _BLK
_VPB = _BLK // 16
_CHB = 25
_CH = _CHB * _BLK
_NCH = _NBLK // _CHB
_NW = 32
_BPW = _BSZ // _NW
_NEG = _np.float32(-_np.inf)
_IMAX = _np.int32(2**31 - 1)


def _sc_body(lp_hbm, sc_hbm,
             o_cs, o_ci, o_cb, o_as, o_ai, o_ab,
             buf, scores_v, bm_ref, cand_ref, mb_ref, sel_ref, gb_ref,
             st_cs, st_ci, st_cb, st_as, st_ai, st_ab):
  cid = lax.axis_index("c")
  sid = lax.axis_index("s")
  w = sid * 2 + cid
  lanes = lax.iota(jnp.int32, 16)
  lane0 = lanes == 0

  def _sget(ref, idx):
    return plsc.load_gather(ref, [jnp.full((16,), idx, jnp.int32)])[0]

  def _sput(ref, idx, val):
    plsc.store_scatter(ref, [jnp.full((16,), idx, jnp.int32)],
                       jnp.full((16,), val), mask=lane0)

  pltpu.sync_copy(sc_hbm, scores_v)

  def per_batch(b2, _):
    batch = w * _BPW + b2

    bm_ref[pl.ds(496, 16)] = jnp.full((16,), _NEG, jnp.float32)

    def per_row(beam, _):
      row = batch * _BEAM + beam
      off_s = _sget(scores_v, row)

      def per_chunk(c, _):
        pltpu.sync_copy(lp_hbm.at[pl.ds(row * _VOCAB + c * _CH, _CH)], buf)

        def per_blk(jb, _):
          base = jb * _BLK
          first = jnp.logical_and(c == 0, jb == 0)
          v0 = buf[pl.ds(base, 16)]
          v0 = jnp.where(jnp.logical_and(first, lanes == _PAD), _NEG, v0)
          acc = v0
          for v in range(1, _VPB):
            acc = jnp.maximum(acc, buf[pl.ds(base + v * 16, 16)])
          _sput(bm_ref, beam * _NBLK + c * _CHB + jb, jnp.max(acc) + off_s)
          return 0

        lax.fori_loop(0, _CHB, per_blk, 0)
        return 0

      lax.fori_loop(0, _NCH, per_chunk, 0)
      return 0

    lax.fori_loop(0, _BEAM, per_row, 0)

    def select(k, _):
      def mx(j, mv):
        return jnp.maximum(mv, bm_ref[pl.ds(j * 16, 16)])

      m = jnp.max(lax.fori_loop(0, 32, mx, jnp.full((16,), _NEG, jnp.float32)))

      def am(j, iv):
        v = bm_ref[pl.ds(j * 16, 16)]
        return jnp.minimum(iv, jnp.where(v == m, lanes + j * 16, _IMAX))

      gb = jnp.min(lax.fori_loop(0, 32, am, jnp.full((16,), _IMAX, jnp.int32)))
      _sput(sel_ref, k, gb)
      _sput(bm_ref, gb, _NEG)
      return 0

    lax.fori_loop(0, _CAND, select, 0)

    selv = jnp.where(lanes < _CAND, sel_ref[...], _IMAX)
    sorted_sel = plsc.sort_key_val(selv, selv)
    if isinstance(sorted_sel, (list, tuple)):
      sorted_sel = sorted_sel[-1]
    sel_ref[...] = sorted_sel

    def gather(s, _):
      gb = _sget(sel_ref, s)
      beam_s = ((gb >= _NBLK).astype(jnp.int32)
                + (gb >= 2 * _NBLK).astype(jnp.int32)
                + (gb >= 3 * _NBLK).astype(jnp.int32))
      blk_s = gb - beam_s * _NBLK
      row_s = batch * _BEAM + beam_s
      _sput(gb_ref, s, beam_s * _VOCAB + blk_s * _BLK)
      pltpu.sync_copy(lp_hbm.at[pl.ds(row_s * _VOCAB + blk_s * _BLK, _BLK)],
                      cand_ref.at[pl.ds(s * _BLK, _BLK)])
      off = _sget(scores_v, row_s)

      def addoff(v, macc):
        x = cand_ref[pl.ds(s * _BLK + v * 16, 16)] + off
        x = jnp.where(
            jnp.logical_and(jnp.logical_and(blk_s == 0, v == 0), lanes == _PAD),
            _NEG, x)
        cand_ref[pl.ds(s * _BLK + v * 16, 16)] = x
        return jnp.maximum(macc, x)

      macc = lax.fori_loop(0, _VPB, addoff, jnp.full((16,), _NEG, jnp.float32))
      mb_ref[pl.ds(s * 16, 16)] = macc
      return 0

    lax.fori_loop(0, _CAND, gather, 0)

    def extract(k, _):
      mv = mb_ref[pl.ds(0, 16)]
      for s in range(1, _CAND):
        mv = jnp.maximum(mv, mb_ref[pl.ds(s * 16, 16)])
      m = jnp.max(mv)
      sstar = jnp.int32(99)
      for s in range(_CAND - 1, -1, -1):
        anyeq = jnp.max(jnp.where(mb_ref[pl.ds(s * 16, 16)] == m, 1, 0))
        sstar = jnp.where(anyeq > 0, jnp.int32(s), sstar)

      def am(v, iv):
        x = cand_ref[pl.ds(sstar * _BLK + v * 16, 16)]
        return jnp.minimum(iv, jnp.where(x == m, lanes + v * 16, _IMAX))

      pos = jnp.min(lax.fori_loop(0, _VPB, am, jnp.full((16,), _IMAX, jnp.int32)))
      g = _sget(gb_ref, sstar) + pos
      _sput(cand_ref, sstar * _BLK + pos, _NEG)

      def remax(v, macc):
        return jnp.maximum(macc, cand_ref[pl.ds(sstar * _BLK + v * 16, 16)])

      mb_ref[pl.ds(sstar * 16, 16)] = lax.fori_loop(
          0, _VPB, remax, jnp.full((16,), _NEG, jnp.float32))

      beam_k = ((g >= _VOCAB).astype(jnp.int32)
                + (g >= 2 * _VOCAB).astype(jnp.int32)
                + (g >= 3 * _VOCAB).astype(jnp.int32))
      _sput(st_cs, b2 * 8 + k, m)
      _sput(st_ci, b2 * 8 + k, g - beam_k * _VOCAB)
      _sput(st_cb, b2 * 8 + k, beam_k)
      return 0

    lax.fori_loop(0, _CAND, extract, 0)
    return 0

  lax.fori_loop(0, _BPW, per_batch, 0)

  tok_vec = st_ci[...]
  is_eos = (tok_vec == _EOS).astype(jnp.int32)
  k_l = lanes & 7
  b2_l = lanes >> 3
  key = b2_l * 100 + is_eos * 8 + k_l
  hyp = plsc.sort_key_val(key, lanes)
  if isinstance(hyp, (list, tuple)):
    hyp = hyp[-1]
  valid = k_l < _BEAM
  as_full = plsc.load_gather(st_cs, [hyp])
  ai_full = plsc.load_gather(st_ci, [hyp])
  ab_full = plsc.load_gather(st_cb, [hyp])
  batch_vec = (w * _BPW + b2_l) * _BEAM
  st_as[...] = jnp.where(valid, as_full, jnp.float32(0.0))
  st_ai[...] = jnp.where(valid, ai_full, jnp.int32(0))
  st_ab[...] = jnp.where(valid, ab_full + batch_vec, jnp.int32(0))

  base = 16 * w
  pltpu.sync_copy(st_cs, o_cs.at[pl.ds(base, 16)])
  pltpu.sync_copy(st_ci, o_ci.at[pl.ds(base, 16)])
  pltpu.sync_copy(st_cb, o_cb.at[pl.ds(base, 16)])
  pltpu.sync_copy(st_as, o_as.at[pl.ds(base, 16)])
  pltpu.sync_copy(st_ai, o_ai.at[pl.ds(base, 16)])
  pltpu.sync_copy(st_ab, o_ab.at[pl.ds(base, 16)])


def _make_launch(interpret=False):
  return functools.partial(
      pl.kernel,
      mesh=plsc.VectorSubcoreMesh(core_axis_name="c", subcore_axis_name="s",
                                  num_cores=2, num_subcores=16),
      out_type=[
          jax.ShapeDtypeStruct((_BSZ * 8,), jnp.float32),
          jax.ShapeDtypeStruct((_BSZ * 8,), jnp.int32),
          jax.ShapeDtypeStruct((_BSZ * 8,), jnp.int32),
          jax.ShapeDtypeStruct((_BSZ * 8,), jnp.float32),
          jax.ShapeDtypeStruct((_BSZ * 8,), jnp.int32),
          jax.ShapeDtypeStruct((_BSZ * 8,), jnp.int32),
      ],
      scratch_types=[
          pltpu.VMEM((_CH,), jnp.float32),
          pltpu.VMEM((256,), jnp.float32),
          pltpu.VMEM((512,), jnp.float32),
          pltpu.VMEM((_CAND * _BLK,), jnp.float32),
          pltpu.VMEM((128,), jnp.float32),
          pltpu.VMEM((16,), jnp.int32),
          pltpu.VMEM((16,), jnp.int32),
          pltpu.VMEM((16,), jnp.float32),
          pltpu.VMEM((16,), jnp.int32),
          pltpu.VMEM((16,), jnp.int32),
          pltpu.VMEM((16,), jnp.float32),
          pltpu.VMEM((16,), jnp.int32),
          pltpu.VMEM((16,), jnp.int32),
      ],
      compiler_params=pltpu.CompilerParams(needs_layout_passes=False),
      interpret=interpret,
  )(_sc_body)


@jax.jit
def kernel(lprobs, scores_prev):
  cs, ci, cb, as_, ai, ab = _make_launch()(lprobs.reshape(-1), scores_prev)
  return (cs.reshape(_BSZ, 8), ci.reshape(_BSZ, 8), cb.reshape(_BSZ, 8),
          as_.reshape(_BSZ, 8)[:, :_BEAM], ai.reshape(_BSZ, 8)[:, :_BEAM],
          ab.reshape(_BSZ, 8)[:, :_BEAM])


# speedup vs baseline: 4.8067x; 1.1457x over previous
import functools

import numpy as _np

import jax
import jax.numpy as jnp
from jax import lax
from jax.experimental import pallas as pl
from jax.experimental.pallas import tpu as pltpu
from jax.experimental.pallas import tpu_sc as plsc

_BSZ = 64
_BEAM = 4
_VOCAB = 100000
_PAD = 1
_EOS = 2
_CAND = 8
_BLK = 2000
_VPB = _BLK // 16
_NBLK = _VOCAB // _BLK
_NBPR = 64
_SUB = 400
_VPS = _SUB // 16
_NSUB = _BLK // _SUB
_CHSZ = 16 * _BLK
_TAILSZ = 2 * _BLK
_NW = 32
_BPW = _BSZ // _NW
_NSTEP = _BPW * _BEAM * 4
_NEG = _np.float32(-_np.inf)
_IMAX = _np.int32(2**31 - 1)


def _sc_body(lp_hbm, sc_hbm,
             o_cs, o_ci, o_cb, o_as, o_ai, o_ab,
             buf, scores_v, bm_ref, cand_ref, mb_ref, sel_ref, gb_ref,
             st_cs, st_ci, st_cb, st_as, st_ai, st_ab,
             sem_a, sem_b):
  cid = lax.axis_index("c")
  sid = lax.axis_index("s")
  w = sid * 2 + cid
  lanes = lax.iota(jnp.int32, 16)
  lane0 = lanes == 0

  def _sget(ref, idx):
    return plsc.load_gather(ref, [jnp.full((16,), idx, jnp.int32)])[0]

  def _sput(ref, idx, val):
    plsc.store_scatter(ref, [jnp.full((16,), idx, jnp.int32)],
                       jnp.full((16,), val), mask=lane0)

  def _treemax(accs):
    while len(accs) > 1:
      accs = [jnp.maximum(accs[i], accs[i + 1]) if i + 1 < len(accs)
              else accs[i] for i in range(0, len(accs), 2)]
    return accs[0]

  pltpu.sync_copy(sc_hbm, scores_v)

  def _chunk_copy(T, start):
    b2 = T >> 4
    t = T & 15
    beam = t >> 2
    c = t & 3
    row = (w * _BPW + b2) * _BEAM + beam
    off = row * _VOCAB + c * _CHSZ
    p = T & 1
    dst = p * _CHSZ
    tail = c == 3

    def go(sem):
      @pl.when(tail)
      def _():
        cp = pltpu.make_async_copy(lp_hbm.at[pl.ds(off, _TAILSZ)],
                                   buf.at[pl.ds(dst, _TAILSZ)], sem)
        cp.start() if start else cp.wait()

      @pl.when(jnp.logical_not(tail))
      def _():
        cp = pltpu.make_async_copy(lp_hbm.at[pl.ds(off, _CHSZ)],
                                   buf.at[pl.ds(dst, _CHSZ)], sem)
        cp.start() if start else cp.wait()

    @pl.when(p == 0)
    def _():
      go(sem_a)

    @pl.when(p == 1)
    def _():
      go(sem_b)

  pltpu.make_async_copy(lp_hbm.at[pl.ds(w * _BPW * _BEAM * _VOCAB, _CHSZ)],
                        buf.at[pl.ds(0, _CHSZ)], sem_a).start()

  def phase_a(T, _):
    nxt = T + 1

    @pl.when(nxt < _NSTEP)
    def _():
      _chunk_copy(nxt, start=True)

    _chunk_copy(T, start=False)

    b2 = T >> 4
    t = T & 15
    beam = t >> 2
    c = t & 3
    row = (w * _BPW + b2) * _BEAM + beam
    off_s = _sget(scores_v, row)
    p = T & 1
    nb = jnp.where(c == 3, 2, 16)

    def blk(i, mv):
      base = p * _CHSZ + i * _BLK
      v0 = buf[pl.ds(base, 16)]
      v0 = jnp.where(
          jnp.logical_and(jnp.logical_and(c == 0, i == 0), lanes == _PAD),
          _NEG, v0)
      accs = [v0] + [buf[pl.ds(base + j * 16, 16)] for j in range(1, 5)]
      for v in range(5, _VPB):
        accs[v % 5] = jnp.maximum(accs[v % 5], buf[pl.ds(base + v * 16, 16)])
      m = jnp.max(_treemax(accs)) + off_s
      return jnp.where(lanes == i, m, mv)

    m_vec = lax.fori_loop(0, nb, blk, jnp.full((16,), _NEG, jnp.float32))
    bm_ref[pl.ds(b2 * 256 + beam * _NBPR + c * 16, 16)] = m_vec
    return 0

  lax.fori_loop(0, _NSTEP, phase_a, 0)

  def phase_b(b2, _):
    batch = w * _BPW + b2
    bmo = b2 * 256

    def select(k, _):
      mvs = [bm_ref[pl.ds(bmo + j * 16, 16)] for j in range(16)]
      m = jnp.max(_treemax(mvs))
      iv = jnp.full((16,), _IMAX, jnp.int32)
      for j in range(16):
        v = bm_ref[pl.ds(bmo + j * 16, 16)]
        iv = jnp.minimum(iv, jnp.where(v == m, lanes + j * 16, _IMAX))
      gb = jnp.min(iv)
      _sput(sel_ref, k, gb)
      _sput(bm_ref, bmo + gb, _NEG)
      return 0

    lax.fori_loop(0, _CAND, select, 0)

    selv = jnp.where(lanes < _CAND, sel_ref[...], _IMAX)
    sorted_sel = plsc.sort_key_val(selv, selv)
    if isinstance(sorted_sel, (list, tuple)):
      sorted_sel = sorted_sel[-1]
    sel_ref[...] = sorted_sel

    def fire(s, _):
      gb = _sget(sel_ref, s)
      beam_s = gb >> 6
      blk_s = gb & 63
      row_s = batch * _BEAM + beam_s
      _sput(gb_ref, s, beam_s * _VOCAB + blk_s * _BLK)
      pltpu.make_async_copy(
          lp_hbm.at[pl.ds(row_s * _VOCAB + blk_s * _BLK, _BLK)],
          cand_ref.at[pl.ds(s * _BLK, _BLK)], sem_a).start()
      return 0

    lax.fori_loop(0, _CAND, fire, 0)

    def drain(s, _):
      pltpu.make_async_copy(lp_hbm.at[pl.ds(0, _BLK)],
                            cand_ref.at[pl.ds(s * _BLK, _BLK)], sem_a).wait()
      return 0

    lax.fori_loop(0, _CAND, drain, 0)

    def mbfill(s, _):
      gb = _sget(sel_ref, s)
      blk_s = gb & 63

      @pl.when(blk_s == 0)
      def _():
        _sput(cand_ref, s * _BLK + _PAD, _NEG)

      off = _sget(scores_v, batch * _BEAM + (gb >> 6))
      for sub in range(_NSUB):
        base = s * _BLK + sub * _SUB
        accs = [cand_ref[pl.ds(base + j * 16, 16)] for j in range(5)]
        for v in range(5, _VPS):
          accs[v % 5] = jnp.maximum(accs[v % 5],
                                    cand_ref[pl.ds(base + v * 16, 16)])
        mb_ref[pl.ds((s * _NSUB + sub) * 16, 16)] = _treemax(accs) + off
      return 0

    lax.fori_loop(0, _CAND, mbfill, 0)

    def extract(k, _):
      mvs = [mb_ref[pl.ds(j * 16, 16)] for j in range(_CAND * _NSUB)]
      m = jnp.max(_treemax(mvs))
      jv = jnp.full((16,), _IMAX, jnp.int32)
      for j in range(_CAND * _NSUB):
        eq = mb_ref[pl.ds(j * 16, 16)] == m
        jv = jnp.minimum(jv, jnp.where(eq, jnp.int32(j), _IMAX))
      jstar = jnp.min(jv)
      sstar = jnp.int32(0)
      for tt in range(1, _CAND):
        sstar = sstar + (jstar >= _NSUB * tt).astype(jnp.int32)
      sub = jstar - _NSUB * sstar
      gbs = _sget(sel_ref, sstar)
      off = _sget(scores_v, batch * _BEAM + (gbs >> 6))
      base = sstar * _BLK + sub * _SUB
      iv = jnp.full((16,), _IMAX, jnp.int32)
      for v in range(_VPS):
        x = cand_ref[pl.ds(base + v * 16, 16)] + off
        iv = jnp.minimum(iv, jnp.where(x == m, lanes + v * 16, _IMAX))
      pos = jnp.min(iv)
      g = _sget(gb_ref, sstar) + sub * _SUB + pos
      _sput(cand_ref, base + pos, _NEG)
      accs = [cand_ref[pl.ds(base + j * 16, 16)] for j in range(5)]
      for v in range(5, _VPS):
        accs[v % 5] = jnp.maximum(accs[v % 5],
                                  cand_ref[pl.ds(base + v * 16, 16)])
      mb_ref[pl.ds(jstar * 16, 16)] = _treemax(accs) + off

      beam_k = ((g >= _VOCAB).astype(jnp.int32)
                + (g >= 2 * _VOCAB).astype(jnp.int32)
                + (g >= 3 * _VOCAB).astype(jnp.int32))
      _sput(st_cs, b2 * 8 + k, m)
      _sput(st_ci, b2 * 8 + k, g - beam_k * _VOCAB)
      _sput(st_cb, b2 * 8 + k, beam_k)
      return 0

    lax.fori_loop(0, _CAND, extract, 0)
    return 0

  lax.fori_loop(0, _BPW, phase_b, 0)

  tok_vec = st_ci[...]
  is_eos = (tok_vec == _EOS).astype(jnp.int32)
  k_l = lanes & 7
  b2_l = lanes >> 3
  key = b2_l * 100 + is_eos * 8 + k_l
  hyp = plsc.sort_key_val(key, lanes)
  if isinstance(hyp, (list, tuple)):
    hyp = hyp[-1]
  valid = k_l < _BEAM
  as_full = plsc.load_gather(st_cs, [hyp])
  ai_full = plsc.load_gather(st_ci, [hyp])
  ab_full = plsc.load_gather(st_cb, [hyp])
  batch_vec = (w * _BPW + b2_l) * _BEAM
  st_as[...] = jnp.where(valid, as_full, jnp.float32(0.0))
  st_ai[...] = jnp.where(valid, ai_full, jnp.int32(0))
  st_ab[...] = jnp.where(valid, ab_full + batch_vec, jnp.int32(0))

  base = 16 * w
  pltpu.sync_copy(st_cs, o_cs.at[pl.ds(base, 16)])
  pltpu.sync_copy(st_ci, o_ci.at[pl.ds(base, 16)])
  pltpu.sync_copy(st_cb, o_cb.at[pl.ds(base, 16)])
  pltpu.sync_copy(st_as, o_as.at[pl.ds(base, 16)])
  pltpu.sync_copy(st_ai, o_ai.at[pl.ds(base, 16)])
  pltpu.sync_copy(st_ab, o_ab.at[pl.ds(base, 16)])


def _make_launch(interpret=False):
  return functools.partial(
      pl.kernel,
      mesh=plsc.VectorSubcoreMesh(core_axis_name="c", subcore_axis_name="s",
                                  num_cores=2, num_subcores=16),
      out_type=[
          jax.ShapeDtypeStruct((_BSZ * 8,), jnp.float32),
          jax.ShapeDtypeStruct((_BSZ * 8,), jnp.int32),
          jax.ShapeDtypeStruct((_BSZ * 8,), jnp.int32),
          jax.ShapeDtypeStruct((_BSZ * 8,), jnp.float32),
          jax.ShapeDtypeStruct((_BSZ * 8,), jnp.int32),
          jax.ShapeDtypeStruct((_BSZ * 8,), jnp.int32),
      ],
      scratch_types=[
          pltpu.VMEM((2 * _CHSZ,), jnp.float32),
          pltpu.VMEM((256,), jnp.float32),
          pltpu.VMEM((512,), jnp.float32),
          pltpu.VMEM((_CAND * _BLK,), jnp.float32),
          pltpu.VMEM((_CAND * _NSUB * 16,), jnp.float32),
          pltpu.VMEM((16,), jnp.int32),
          pltpu.VMEM((16,), jnp.int32),
          pltpu.VMEM((16,), jnp.float32),
          pltpu.VMEM((16,), jnp.int32),
          pltpu.VMEM((16,), jnp.int32),
          pltpu.VMEM((16,), jnp.float32),
          pltpu.VMEM((16,), jnp.int32),
          pltpu.VMEM((16,), jnp.int32),
          pltpu.SemaphoreType.DMA,
          pltpu.SemaphoreType.DMA,
      ],
      compiler_params=pltpu.CompilerParams(needs_layout_passes=False),
      interpret=interpret,
  )(_sc_body)


@jax.jit
def kernel(lprobs, scores_prev):
  cs, ci, cb, as_, ai, ab = _make_launch()(lprobs.reshape(-1), scores_prev)
  return (cs.reshape(_BSZ, 8), ci.reshape(_BSZ, 8), cb.reshape(_BSZ, 8),
          as_.reshape(_BSZ, 8)[:, :_BEAM], ai.reshape(_BSZ, 8)[:, :_BEAM],
          ab.reshape(_BSZ, 8)[:, :_BEAM])


# speedup vs baseline: 8.5799x; 1.7850x over previous
import functools

import numpy as _np

import jax
import jax.numpy as jnp
from jax import lax
from jax.experimental import pallas as pl
from jax.experimental.pallas import tpu as pltpu
from jax.experimental.pallas import tpu_sc as plsc

_BSZ = 64
_BEAM = 4
_VOCAB = 100000
_PAD = 1
_EOS = 2
_CAND = 8
_BLK = 512
_VPB = _BLK // 16
_NBLK = 196
_NBPR = 256
_CC = 4096
_NFULL = 24
_TA_OFF = _NFULL * _CC
_TA_W = 1664
_TB_OFF = 99968
_TB_W = 32
_NW = 32
_BPW = _BSZ // _NW
_NEG = _np.float32(-_np.inf)
_IMAX = _np.int32(2**31 - 1)


def _sc_body(lp_hbm, ta_hbm, tb_hbm, sc_hbm,
             o_cs, o_ci, o_cb, o_as, o_ai, o_ab,
             buf_a, buf_b, tbuf_a, tbuf_b, scores_v, bm_ref, cand3,
             mb_ref, sel_ref, gb_ref,
             st_cs, st_ci, st_cb, st_as, st_ai, st_ab,
             sem_a, sem_b, sem_c):
  cid = lax.axis_index("c")
  sid = lax.axis_index("s")
  w = sid * 2 + cid
  row0 = w * 8
  lanes = lax.iota(jnp.int32, 16)
  lane0 = lanes == 0

  def _sget(ref, idx):
    return plsc.load_gather(ref, [jnp.full((16,), idx, jnp.int32)])[0]

  def _sput(ref, idx, val):
    plsc.store_scatter(ref, [jnp.full((16,), idx, jnp.int32)],
                       jnp.full((16,), val), mask=lane0)

  def _treemax(accs):
    while len(accs) > 1:
      accs = [jnp.maximum(accs[i], accs[i + 1]) if i + 1 < len(accs)
              else accs[i] for i in range(0, len(accs), 2)]
    return accs[0]

  def _chainmax(load, n):
    k = min(5, n)
    accs = [load(v) for v in range(k)]
    for v in range(k, n):
      accs[v % k] = jnp.maximum(accs[v % k], load(v))
    return _treemax(accs)

  pltpu.sync_copy(sc_hbm, scores_v)

  pltpu.make_async_copy(lp_hbm.at[pl.ds(row0, 8), pl.ds(0, _CC)],
                        buf_a, sem_a).start()
  pltpu.make_async_copy(ta_hbm.at[pl.ds(row0, 8)], tbuf_a, sem_c).start()
  pltpu.make_async_copy(tb_hbm.at[pl.ds(row0, 8)], tbuf_b, sem_c).start()

  def _process_chunk(buf, t):
    def per_pair(g, _):
      def per_u(u, mv):
        r = 2 * g + (u >> 3)
        j = u & 7

        base_j = j * _BLK
        v0 = buf[r, pl.ds(base_j, 16)]
        first = jnp.logical_and(t == 0, j == 0)
        v0 = jnp.where(jnp.logical_and(first, lanes == _PAD), _NEG, v0)
        k = 5
        accs = [v0] + [buf[r, pl.ds(base_j + v * 16, 16)] for v in range(1, k)]
        for v in range(k, _VPB):
          accs[v % k] = jnp.maximum(accs[v % k],
                                    buf[r, pl.ds(base_j + v * 16, 16)])
        off = _sget(scores_v, row0 + r)
        m = jnp.max(_treemax(accs)) + off
        return jnp.where(lanes == u, m, mv)

      mv = lax.fori_loop(0, 16, per_u, jnp.full((16,), _NEG, jnp.float32))
      r_vec = 2 * g + (lanes >> 3)
      idx = (r_vec << 8) + t * 8 + (lanes & 7)
      plsc.store_scatter(bm_ref, [idx], mv)
      return 0

    lax.fori_loop(0, 4, per_pair, 0)

  def _issue(buf, t, sem):
    pltpu.make_async_copy(lp_hbm.at[pl.ds(row0, 8), pl.ds(t * _CC, _CC)],
                          buf, sem).start()

  def _wait(buf, sem):
    pltpu.make_async_copy(lp_hbm.at[pl.ds(0, 8), pl.ds(0, _CC)],
                          buf, sem).wait()

  def phase_a(i, _):
    ta = 2 * i
    tb = 2 * i + 1

    @pl.when(tb < _NFULL)
    def _():
      _issue(buf_b, tb, sem_b)

    _wait(buf_a, sem_a)
    _process_chunk(buf_a, ta)

    @pl.when(ta + 2 < _NFULL)
    def _():
      _issue(buf_a, ta + 2, sem_a)

    @pl.when(tb < _NFULL)
    def _():
      _wait(buf_b, sem_b)
      _process_chunk(buf_b, tb)

    return 0

  lax.fori_loop(0, _NFULL // 2, phase_a, 0)

  pltpu.make_async_copy(ta_hbm.at[pl.ds(0, 8)], tbuf_a, sem_c).wait()
  pltpu.make_async_copy(tb_hbm.at[pl.ds(0, 8)], tbuf_b, sem_c).wait()

  def tails(r, _):
    off = _sget(scores_v, row0 + r)

    def tblk(j, _):
      m = _chainmax(lambda v: tbuf_a[r, pl.ds(j * _BLK + v * 16, 16)], _VPB)
      _sput(bm_ref, (r << 8) + 192 + j, jnp.max(m) + off)
      return 0

    lax.fori_loop(0, 3, tblk, 0)
    p1 = _chainmax(lambda v: tbuf_a[r, pl.ds(3 * _BLK + v * 16, 16)], 8)
    p2 = jnp.maximum(tbuf_b[r, pl.ds(0, 16)], tbuf_b[r, pl.ds(16, 16)])
    _sput(bm_ref, (r << 8) + 195, jnp.max(jnp.maximum(p1, p2)) + off)
    return 0

  lax.fori_loop(0, 8, tails, 0)

  def vfill(i, _):
    r = i >> 2
    qq = i & 3
    base = (r << 8) + 192 + qq * 16
    cur = bm_ref[pl.ds(base, 16)]
    keep = jnp.logical_and(qq == 0, lanes < 4)
    bm_ref[pl.ds(base, 16)] = jnp.where(keep, cur, _NEG)
    return 0

  lax.fori_loop(0, 32, vfill, 0)

  def phase_b(b2, _):
    batch = w * _BPW + b2
    bmo = b2 * 1024

    def select(k, _):
      def mx(jj, mv):
        return jnp.maximum(mv, bm_ref[pl.ds(bmo + jj * 16, 16)])

      m = jnp.max(lax.fori_loop(0, 64, mx, jnp.full((16,), _NEG, jnp.float32)))
      def am(jj, iv):
        v = bm_ref[pl.ds(bmo + jj * 16, 16)]
        return jnp.minimum(iv, jnp.where(v == m, lanes + jj * 16, _IMAX))

      gb = jnp.min(lax.fori_loop(0, 64, am, jnp.full((16,), _IMAX, jnp.int32)))
      _sput(sel_ref, k, gb)
      _sput(bm_ref, bmo + gb, _NEG)
      return 0

    lax.fori_loop(0, _CAND, select, 0)

    selv = jnp.where(lanes < _CAND, sel_ref[...], _IMAX)
    sorted_sel = plsc.sort_key_val(selv, selv)
    if isinstance(sorted_sel, (list, tuple)):
      sorted_sel = sorted_sel[-1]
    sel_ref[...] = sorted_sel

    def _splat(x):
      return jnp.full((16,), x, jnp.int32)

    def _bload(s, r_s, v):
      return plsc.load_gather(cand3, [_splat(s), _splat(r_s), v * 16 + lanes])

    def fire(s, _):
      gb = _sget(sel_ref, s)
      beam_s = gb >> 8
      cb = gb & 255
      r_s = b2 * _BEAM + beam_s
      _sput(gb_ref, s, beam_s * _VOCAB + cb * _BLK)

      @pl.when(cb < 195)
      def _():
        pltpu.make_async_copy(
            lp_hbm.at[pl.ds(row0, 8), pl.ds(cb * _BLK, _BLK)],
            cand3.at[s], sem_a).start()

      @pl.when(cb == 195)
      def _():
        for v in range(8):
          x = plsc.load_gather(tbuf_a, [_splat(r_s), 1536 + v * 16 + lanes])
          plsc.store_scatter(cand3, [_splat(s), _splat(r_s), v * 16 + lanes], x)
        for v in range(2):
          x = plsc.load_gather(tbuf_b, [_splat(r_s), v * 16 + lanes])
          plsc.store_scatter(cand3,
                             [_splat(s), _splat(r_s), 128 + v * 16 + lanes], x)
      return 0

    lax.fori_loop(0, _CAND, fire, 0)

    def drain(s, _):
      gb = _sget(sel_ref, s)
      cb = gb & 255

      @pl.when(cb < 195)
      def _():
        pltpu.make_async_copy(lp_hbm.at[pl.ds(0, 8), pl.ds(0, _BLK)],
                              cand3.at[s], sem_a).wait()
      return 0

    lax.fori_loop(0, _CAND, drain, 0)

    def mbfill(s, _):
      gb = _sget(sel_ref, s)
      cb = gb & 255
      r_s = b2 * _BEAM + (gb >> 8)

      @pl.when(cb == 0)
      def _():
        plsc.store_scatter(cand3, [_splat(s), _splat(r_s), _splat(_PAD)],
                           jnp.full((16,), _NEG, jnp.float32), mask=lane0)

      @pl.when(cb == 195)
      def _():
        def rfill(v, _):
          plsc.store_scatter(cand3, [_splat(s), _splat(r_s), 160 + v * 16 + lanes],
                             jnp.full((16,), _NEG, jnp.float32))
          return 0
        lax.fori_loop(0, 22, rfill, 0)

      off = _sget(scores_v, batch * _BEAM + (gb >> 8))
      mm = _chainmax(lambda v: _bload(s, r_s, v), _VPB)
      mb_ref[pl.ds(s * 16, 16)] = mm + off
      return 0

    lax.fori_loop(0, _CAND, mbfill, 0)

    def extract(k, _):
      mvs = [mb_ref[pl.ds(j * 16, 16)] for j in range(_CAND)]
      m = jnp.max(_treemax(mvs))
      sstar = jnp.int32(99)
      for s in range(_CAND - 1, -1, -1):
        anyeq = jnp.max(jnp.where(mvs[s] == m, 1, 0))
        sstar = jnp.where(anyeq > 0, jnp.int32(s), sstar)
      gbs = _sget(sel_ref, sstar)
      r_star = b2 * _BEAM + (gbs >> 8)
      off = _sget(scores_v, batch * _BEAM + (gbs >> 8))
      iv = jnp.full((16,), _IMAX, jnp.int32)
      for v in range(_VPB):
        x = _bload(sstar, r_star, v) + off
        iv = jnp.minimum(iv, jnp.where(x == m, lanes + v * 16, _IMAX))
      pos = jnp.min(iv)
      g = _sget(gb_ref, sstar) + pos
      plsc.store_scatter(cand3, [_splat(sstar), _splat(r_star), _splat(pos)],
                         jnp.full((16,), _NEG, jnp.float32), mask=lane0)
      mm = _chainmax(lambda v: _bload(sstar, r_star, v), _VPB)
      mb_ref[pl.ds(sstar * 16, 16)] = mm + off

      beam_k = ((g >= _VOCAB).astype(jnp.int32)
                + (g >= 2 * _VOCAB).astype(jnp.int32)
                + (g >= 3 * _VOCAB).astype(jnp.int32))
      _sput(st_cs, b2 * 8 + k, m)
      _sput(st_ci, b2 * 8 + k, g - beam_k * _VOCAB)
      _sput(st_cb, b2 * 8 + k, beam_k)
      return 0

    lax.fori_loop(0, _CAND, extract, 0)
    return 0

  lax.fori_loop(0, _BPW, phase_b, 0)

  tok_vec = st_ci[...]
  is_eos = (tok_vec == _EOS).astype(jnp.int32)
  k_l = lanes & 7
  b2_l = lanes >> 3
  key = b2_l * 100 + is_eos * 8 + k_l
  hyp = plsc.sort_key_val(key, lanes)
  if isinstance(hyp, (list, tuple)):
    hyp = hyp[-1]
  valid = k_l < _BEAM
  as_full = plsc.load_gather(st_cs, [hyp])
  ai_full = plsc.load_gather(st_ci, [hyp])
  ab_full = plsc.load_gather(st_cb, [hyp])
  batch_vec = (w * _BPW + b2_l) * _BEAM
  st_as[...] = jnp.where(valid, as_full, jnp.float32(0.0))
  st_ai[...] = jnp.where(valid, ai_full, jnp.int32(0))
  st_ab[...] = jnp.where(valid, ab_full + batch_vec, jnp.int32(0))

  base = 16 * w
  pltpu.sync_copy(st_cs, o_cs.at[pl.ds(base, 16)])
  pltpu.sync_copy(st_ci, o_ci.at[pl.ds(base, 16)])
  pltpu.sync_copy(st_cb, o_cb.at[pl.ds(base, 16)])
  pltpu.sync_copy(st_as, o_as.at[pl.ds(base, 16)])
  pltpu.sync_copy(st_ai, o_ai.at[pl.ds(base, 16)])
  pltpu.sync_copy(st_ab, o_ab.at[pl.ds(base, 16)])


def _make_launch(interpret=False):
  return functools.partial(
      pl.kernel,
      mesh=plsc.VectorSubcoreMesh(core_axis_name="c", subcore_axis_name="s",
                                  num_cores=2, num_subcores=16),
      out_type=[
          jax.ShapeDtypeStruct((_BSZ * 8,), jnp.float32),
          jax.ShapeDtypeStruct((_BSZ * 8,), jnp.int32),
          jax.ShapeDtypeStruct((_BSZ * 8,), jnp.int32),
          jax.ShapeDtypeStruct((_BSZ * 8,), jnp.float32),
          jax.ShapeDtypeStruct((_BSZ * 8,), jnp.int32),
          jax.ShapeDtypeStruct((_BSZ * 8,), jnp.int32),
      ],
      scratch_types=[
          pltpu.VMEM((8, _CC), jnp.float32),
          pltpu.VMEM((8, _CC), jnp.float32),
          pltpu.VMEM((8, _TA_W), jnp.float32),
          pltpu.VMEM((8, 128), jnp.float32),
          pltpu.VMEM((256,), jnp.float32),
          pltpu.VMEM((2048,), jnp.float32),
          pltpu.VMEM((_CAND, 8, _BLK), jnp.float32),
          pltpu.VMEM((_CAND * 16,), jnp.float32),
          pltpu.VMEM((16,), jnp.int32),
          pltpu.VMEM((16,), jnp.int32),
          pltpu.VMEM((16,), jnp.float32),
          pltpu.VMEM((16,), jnp.int32),
          pltpu.VMEM((16,), jnp.int32),
          pltpu.VMEM((16,), jnp.float32),
          pltpu.VMEM((16,), jnp.int32),
          pltpu.VMEM((16,), jnp.int32),
          pltpu.SemaphoreType.DMA,
          pltpu.SemaphoreType.DMA,
          pltpu.SemaphoreType.DMA,
      ],
      compiler_params=pltpu.CompilerParams(needs_layout_passes=False),
      interpret=interpret,
  )(_sc_body)


@jax.jit
def kernel(lprobs, scores_prev):
  tail_a = lax.slice(lprobs, (0, _TA_OFF), (_BSZ * _BEAM, _TB_OFF))
  tail_b = lax.pad(lax.slice(lprobs, (0, _TB_OFF), (_BSZ * _BEAM, _VOCAB)),
                   _np.float32(-_np.inf), ((0, 0, 0), (0, 96, 0)))
  cs, ci, cb, as_, ai, ab = _make_launch()(lprobs, tail_a, tail_b, scores_prev)
  return (cs.reshape(_BSZ, 8), ci.reshape(_BSZ, 8), cb.reshape(_BSZ, 8),
          as_.reshape(_BSZ, 8)[:, :_BEAM], ai.reshape(_BSZ, 8)[:, :_BEAM],
          ab.reshape(_BSZ, 8)[:, :_BEAM])


# speedup vs baseline: 12.7821x; 1.4898x over previous
import functools

import numpy as _np

import jax
import jax.numpy as jnp
from jax import lax
from jax.experimental import pallas as pl
from jax.experimental.pallas import tpu as pltpu
from jax.experimental.pallas import tpu_sc as plsc

_BSZ = 64
_BEAM = 4
_VOCAB = 100000
_PAD = 1
_EOS = 2
_CAND = 8
_NT = 782
_NTP = 800
_TPW = 25
_NW = 32
_BPW = _BSZ // _NW
_NEG = _np.float32(-_np.inf)
_IMAX = _np.int32(2**31 - 1)


def _mesh():
  return plsc.VectorSubcoreMesh(core_axis_name="c", subcore_axis_name="s",
                                num_cores=2, num_subcores=16)


def _wid():
  return lax.axis_index("s") * 2 + lax.axis_index("c")


def _treemax(accs):
  while len(accs) > 1:
    accs = [jnp.maximum(accs[i], accs[i + 1]) if i + 1 < len(accs)
            else accs[i] for i in range(0, len(accs), 2)]
  return accs[0]


def _chainmax(load, n):
  k = min(5, n)
  accs = [load(v) for v in range(k)]
  for v in range(k, n):
    accs[v % k] = jnp.maximum(accs[v % k], load(v))
  return _treemax(accs)


def _bm_body(lpt_hbm, o_bm, buf_a, buf_b, bmv, sem_a, sem_b):
  w = _wid()
  t0 = w * _TPW
  ntiles = jnp.where(w == _NW - 1, _NT - (_NW - 1) * _TPW, _TPW)

  def _xfer(buf, ti, sem, start):
    t = t0 + ti

    @pl.when(t == _NT - 1)
    def _():
      cp = pltpu.make_async_copy(lpt_hbm.at[pl.ds(t * 128, 32)],
                                 buf.at[pl.ds(0, 32)], sem)
      cp.start() if start else cp.wait()

    @pl.when(t < _NT - 1)
    def _():
      cp = pltpu.make_async_copy(lpt_hbm.at[pl.ds(t * 128, 128)], buf, sem)
      cp.start() if start else cp.wait()

  def _proc(buf, ti):
    t = t0 + ti

    @pl.when(t == 0)
    def _():
      def padfix(rg, _):
        buf[1, pl.ds(rg * 16, 16)] = jnp.full((16,), _NEG, jnp.float32)
        return 0
      lax.fori_loop(0, 16, padfix, 0)

    @pl.when(t == _NT - 1)
    def _():
      def rg_ragged(rg, _):
        acc = _chainmax(lambda v: buf[v, pl.ds(rg * 16, 16)], 32)
        bmv[pl.ds(rg * 16, 16)] = acc
        return 0
      lax.fori_loop(0, 16, rg_ragged, 0)

    @pl.when(t < _NT - 1)
    def _():
      def rg_full(rg, _):
        acc = _chainmax(lambda v: buf[v, pl.ds(rg * 16, 16)], 128)
        bmv[pl.ds(rg * 16, 16)] = acc
        return 0
      lax.fori_loop(0, 16, rg_full, 0)

    pltpu.sync_copy(bmv, o_bm.at[pl.ds(t * 256, 256)])

  _xfer(buf_a, 0, sem_a, True)

  def pipe(i, _):
    ta = 2 * i
    tb = 2 * i + 1

    @pl.when(tb < ntiles)
    def _():
      _xfer(buf_b, tb, sem_b, True)

    @pl.when(ta < ntiles)
    def _():
      _xfer(buf_a, ta, sem_a, False)
      _proc(buf_a, ta)

    @pl.when(ta + 2 < ntiles)
    def _():
      _xfer(buf_a, ta + 2, sem_a, True)

    @pl.when(tb < ntiles)
    def _():
      _xfer(buf_b, tb, sem_b, False)
      _proc(buf_b, tb)

    return 0

  lax.fori_loop(0, (_TPW + 1) // 2, pipe, 0)

  @pl.when(w == _NW - 1)
  def _():
    def negfill(rg, _):
      bmv[pl.ds(rg * 16, 16)] = jnp.full((16,), _NEG, jnp.float32)
      return 0
    lax.fori_loop(0, 16, negfill, 0)

    def vtile(q, _):
      pltpu.sync_copy(bmv, o_bm.at[pl.ds((_NT + q) * 256, 256)])
      return 0
    lax.fori_loop(0, _NTP - _NT, vtile, 0)


def _sel_body(bmt_hbm, sc_hbm, o_sel, bmv2, scores_v, st_sel):
  u = _wid()
  lanes = lax.iota(jnp.int32, 16)
  lane0 = lanes == 0

  def _sget(ref, idx):
    return plsc.load_gather(ref, [jnp.full((16,), idx, jnp.int32)])[0]

  def _sput(ref, idx, val):
    plsc.store_scatter(ref, [jnp.full((16,), idx, jnp.int32)],
                       jnp.full((16,), val), mask=lane0)

  pltpu.sync_copy(bmt_hbm.at[pl.ds(8 * u, 8)], bmv2)
  pltpu.sync_copy(sc_hbm, scores_v)

  def per_batch(b2, _):
    batch = u * _BPW + b2
    offs = [_sget(scores_v, batch * _BEAM + beam) for beam in range(_BEAM)]

    def select(k, _):
      m = jnp.full((16,), _NEG, jnp.float32)
      for beam in range(_BEAM):
        rl = b2 * _BEAM + beam

        def ld(j, beam=beam, rl=rl):
          return bmv2[rl, pl.ds(j * 16, 16)] + offs[beam]

        m = jnp.maximum(m, _chainmax(ld, _NTP // 16))
      mm = jnp.max(m)
      iv = jnp.full((16,), _IMAX, jnp.int32)
      for beam in range(_BEAM):
        rl = b2 * _BEAM + beam
        for j in range(_NTP // 16):
          v = bmv2[rl, pl.ds(j * 16, 16)] + offs[beam]
          iv = jnp.minimum(
              iv, jnp.where(v == mm, beam * 1024 + j * 16 + lanes, _IMAX))
      gb = jnp.min(iv)
      _sput(st_sel, b2 * 8 + k, gb)
      beam_g = gb >> 10
      t_g = gb & 1023
      plsc.store_scatter(bmv2,
                         [jnp.full((16,), b2 * _BEAM + beam_g, jnp.int32),
                          jnp.full((16,), t_g, jnp.int32)],
                         jnp.full((16,), _NEG, jnp.float32), mask=lane0)
      return 0

    lax.fori_loop(0, _CAND, select, 0)
    return 0

  lax.fori_loop(0, _BPW, per_batch, 0)
  pltpu.sync_copy(st_sel, o_sel.at[pl.ds(16 * u, 16)])


def _ex_body(cand_hbm, sel_hbm, sc_hbm,
             o_cs, o_ci, o_cb, o_as, o_ai, o_ab,
             cbuf, selv, scores_v,
             st_cs, st_ci, st_cb, st_as, st_ai, st_ab):
  u = _wid()
  lanes = lax.iota(jnp.int32, 16)
  lane0 = lanes == 0

  def _sget(ref, idx):
    return plsc.load_gather(ref, [jnp.full((16,), idx, jnp.int32)])[0]

  def _sput(ref, idx, val):
    plsc.store_scatter(ref, [jnp.full((16,), idx, jnp.int32)],
                       jnp.full((16,), val), mask=lane0)

  def _splat(x):
    return jnp.full((16,), x, jnp.int32)

  pltpu.sync_copy(sel_hbm.at[pl.ds(16 * u, 16)], selv)
  pltpu.sync_copy(sc_hbm, scores_v)

  for b2 in range(_BPW):
    batch = u * _BPW + b2
    pltpu.sync_copy(cand_hbm.at[batch], cbuf)

    offs = []
    gbases = []
    for s in range(_CAND):
      gb_s = _sget(selv, b2 * 8 + s)
      beam_s = gb_s >> 10
      t_s = gb_s & 1023
      offs.append(_sget(scores_v, batch * _BEAM + beam_s))
      gbases.append(beam_s * _VOCAB + t_s * 128)

      @pl.when(t_s == 0)
      def _(s=s):
        plsc.store_scatter(cbuf, [_splat(s), _splat(_PAD)],
                           jnp.full((16,), _NEG, jnp.float32), mask=lane0)

      @pl.when(t_s == _NT - 1)
      def _(s=s):
        for v in range(2, 8):
          plsc.store_scatter(cbuf, [_splat(s), v * 16 + lanes],
                             jnp.full((16,), _NEG, jnp.float32))

    def extract(k, _):
      m = jnp.full((16,), _NEG, jnp.float32)
      for s in range(_CAND):
        m = jnp.maximum(
            m, _chainmax(lambda v, s=s: cbuf[s, pl.ds(v * 16, 16)] + offs[s],
                         8))
      mm = jnp.max(m)
      iv = jnp.full((16,), _IMAX, jnp.int32)
      for s in range(_CAND):
        for v in range(8):
          x = cbuf[s, pl.ds(v * 16, 16)] + offs[s]
          iv = jnp.minimum(
              iv, jnp.where(x == mm, gbases[s] + v * 16 + lanes, _IMAX))
      g = jnp.min(iv)
      for s in range(_CAND):
        pos = g - gbases[s]

        @pl.when(jnp.logical_and(pos >= 0, pos < 128))
        def _(s=s, pos=pos):
          plsc.store_scatter(cbuf, [_splat(s), _splat(pos)],
                             jnp.full((16,), _NEG, jnp.float32), mask=lane0)

      beam_k = ((g >= _VOCAB).astype(jnp.int32)
                + (g >= 2 * _VOCAB).astype(jnp.int32)
                + (g >= 3 * _VOCAB).astype(jnp.int32))
      _sput(st_cs, b2 * 8 + k, mm)
      _sput(st_ci, b2 * 8 + k, g - beam_k * _VOCAB)
      _sput(st_cb, b2 * 8 + k, beam_k)
      return 0

    lax.fori_loop(0, _CAND, extract, 0)

  tok_vec = st_ci[...]
  is_eos = (tok_vec == _EOS).astype(jnp.int32)
  k_l = lanes & 7
  b2_l = lanes >> 3
  key = b2_l * 100 + is_eos * 8 + k_l
  hyp = plsc.sort_key_val(key, lanes)
  if isinstance(hyp, (list, tuple)):
    hyp = hyp[-1]
  valid = k_l < _BEAM
  as_full = plsc.load_gather(st_cs, [hyp])
  ai_full = plsc.load_gather(st_ci, [hyp])
  ab_full = plsc.load_gather(st_cb, [hyp])
  batch_vec = (u * _BPW + b2_l) * _BEAM
  st_as[...] = jnp.where(valid, as_full, jnp.float32(0.0))
  st_ai[...] = jnp.where(valid, ai_full, jnp.int32(0))
  st_ab[...] = jnp.where(valid, ab_full + batch_vec, jnp.int32(0))

  base = 16 * u
  pltpu.sync_copy(st_cs, o_cs.at[pl.ds(base, 16)])
  pltpu.sync_copy(st_ci, o_ci.at[pl.ds(base, 16)])
  pltpu.sync_copy(st_cb, o_cb.at[pl.ds(base, 16)])
  pltpu.sync_copy(st_as, o_as.at[pl.ds(base, 16)])
  pltpu.sync_copy(st_ai, o_ai.at[pl.ds(base, 16)])
  pltpu.sync_copy(st_ab, o_ab.at[pl.ds(base, 16)])


def _launch_bm():
  return functools.partial(
      pl.kernel, mesh=_mesh(),
      out_type=[jax.ShapeDtypeStruct((_NTP * 256,), jnp.float32)],
      scratch_types=[
          pltpu.VMEM((128, 256), jnp.float32),
          pltpu.VMEM((128, 256), jnp.float32),
          pltpu.VMEM((256,), jnp.float32),
          pltpu.SemaphoreType.DMA,
          pltpu.SemaphoreType.DMA,
      ],
      compiler_params=pltpu.CompilerParams(needs_layout_passes=False),
  )(_bm_body)


def _launch_sel():
  return functools.partial(
      pl.kernel, mesh=_mesh(),
      out_type=[jax.ShapeDtypeStruct((_BSZ * 8,), jnp.int32)],
      scratch_types=[
          pltpu.VMEM((8, _NTP), jnp.float32),
          pltpu.VMEM((256,), jnp.float32),
          pltpu.VMEM((16,), jnp.int32),
      ],
      compiler_params=pltpu.CompilerParams(needs_layout_passes=False),
  )(_sel_body)


def _launch_ex():
  return functools.partial(
      pl.kernel, mesh=_mesh(),
      out_type=[
          jax.ShapeDtypeStruct((_BSZ * 8,), jnp.float32),
          jax.ShapeDtypeStruct((_BSZ * 8,), jnp.int32),
          jax.ShapeDtypeStruct((_BSZ * 8,), jnp.int32),
          jax.ShapeDtypeStruct((_BSZ * 8,), jnp.float32),
          jax.ShapeDtypeStruct((_BSZ * 8,), jnp.int32),
          jax.ShapeDtypeStruct((_BSZ * 8,), jnp.int32),
      ],
      scratch_types=[
          pltpu.VMEM((_CAND, 128), jnp.float32),
          pltpu.VMEM((16,), jnp.int32),
          pltpu.VMEM((256,), jnp.float32),
          pltpu.VMEM((16,), jnp.float32),
          pltpu.VMEM((16,), jnp.int32),
          pltpu.VMEM((16,), jnp.int32),
          pltpu.VMEM((16,), jnp.float32),
          pltpu.VMEM((16,), jnp.int32),
          pltpu.VMEM((16,), jnp.int32),
      ],
      compiler_params=pltpu.CompilerParams(needs_layout_passes=False),
  )(_ex_body)


@jax.jit
def kernel(lprobs, scores_prev):
  (bm1d,) = _launch_bm()(lprobs.T)
  bmt = bm1d.reshape(_NTP, 256).T
  (sel,) = _launch_sel()(bmt, scores_prev)
  sel64 = sel.reshape(_BSZ, 8)
  beam = sel64 >> 10
  t = sel64 & 1023
  rows = jnp.arange(_BSZ, dtype=jnp.int32)[:, None] * _BEAM + beam
  cols = t[..., None] * 128 + jnp.arange(128, dtype=jnp.int32)[None, None, :]
  cols = jnp.minimum(cols, _VOCAB - 1)
  cand = lprobs[rows[:, :, None], cols]
  cs, ci, cb, as_, ai, ab = _launch_ex()(cand, sel, scores_prev)
  return (cs.reshape(_BSZ, 8), ci.reshape(_BSZ, 8), cb.reshape(_BSZ, 8),
          as_.reshape(_BSZ, 8)[:, :_BEAM], ai.reshape(_BSZ, 8)[:, :_BEAM],
          ab.reshape(_BSZ, 8)[:, :_BEAM])
